# baseline (speedup 1.0000x reference)
import jax
import jax.numpy as jnp
import numpy as np
from jax import lax
from jax.experimental import pallas as pl
from jax.experimental.pallas import tpu as pltpu
from jax.experimental.pallas import tpu_sc as plsc

B = 512
D = 1024
NH = 8
DH = D // NH
E = 16
K = 2
P = B * K
R = 128
NBPE = P // R
CPE = NBPE * R
OPLEN = E * CPE
MLEN = 2 * P + 2 * B + E
SC_NC = 2
SC_NS = 16
NW = SC_NC * SC_NS


def _dott(a, b):
    return lax.dot_general(a, b, (((1,), (1,)), ((), ())),
                           preferred_element_type=jnp.float32)


def _dot(a, b):
    return lax.dot_general(a, b, (((1,), (0,)), ((), ())),
                           preferred_element_type=jnp.float32)


def _gelu(x):
    return 0.5 * x * (1.0 + lax.erf(x * np.float32(1.0 / np.sqrt(2.0))))


def _mega_body(x_ref, pw_ref, pb_ref, iw_ref, ib_ref, ow_ref, ob_ref,
               lg_ref, lb_ref, gw_ref, gb_ref, eb5_ref,
               w1_ref, b1_ref, w2_ref, b2_ref, w3_ref, b3_ref, w4_ref, b4_ref,
               w5_ref,
               op_ref, meta_ref,
               z_s, er_s, cnt_s):
    j = pl.program_id(0)

    @pl.when(j == 0)
    def _front():
        proj = _dott(x_ref[...], pw_ref[...]) + pb_ref[...]
        qkv = _dott(proj, iw_ref[...]) + ib_ref[...]
        heads = []
        for h in range(NH):
            q = qkv[:, h * DH:(h + 1) * DH]
            k = qkv[:, D + h * DH:D + (h + 1) * DH]
            v = qkv[:, 2 * D + h * DH:2 * D + (h + 1) * DH]
            s = _dott(q, k) * np.float32(1.0 / np.sqrt(DH))
            m = jnp.max(s, axis=-1, keepdims=True)
            ex = jnp.exp(s - m)
            attn = ex / jnp.sum(ex, axis=-1, keepdims=True)
            heads.append(_dot(attn, v))
        o = jnp.concatenate(heads, axis=1)
        ao = _dott(o, ow_ref[...]) + ob_ref[...]
        mu = jnp.mean(ao, axis=-1, keepdims=True)
        var = jnp.mean((ao - mu) ** 2, axis=-1, keepdims=True)
        z = ((ao - mu) / jnp.sqrt(var + np.float32(1e-5)) * lg_ref[...]
             + lb_ref[...])
        z_s[...] = z

        g = _dott(z, gw_ref[...]) + gb_ref[...]
        ioL = lax.broadcasted_iota(jnp.int32, (B, E), 1)
        l1 = jnp.max(g, axis=1, keepdims=True)
        i1 = jnp.min(jnp.where(g == l1, ioL, E), axis=1, keepdims=True)
        gm = jnp.where(ioL == i1, -jnp.inf, g)
        l2 = jnp.max(gm, axis=1, keepdims=True)
        i2 = jnp.min(jnp.where(gm == l2, ioL, E), axis=1, keepdims=True)
        w1 = 1.0 / (1.0 + jnp.exp(l2 - l1))
        w2 = 1.0 - w1

        oh = jnp.concatenate([ioL == i1, ioL == i2], axis=0).astype(jnp.float32)
        cnt_s[...] = jnp.sum(oh, axis=0, keepdims=True).astype(jnp.int32)
        tr = lax.broadcasted_iota(jnp.int32, (P, P), 0)
        tc = lax.broadcasted_iota(jnp.int32, (P, P), 1)
        tstrict = (tc < tr).astype(jnp.float32)
        rankmat = _dot(tstrict, oh)
        rank = jnp.sum(rankmat * oh, axis=1,
                       keepdims=True).astype(jnp.int32)
        er_s[...] = jnp.concatenate([i1, i2, rank[:B], rank[B:]], axis=1)

        e_flat = jnp.concatenate([i1, i2], axis=0)
        pos = (e_flat * CPE + rank).astype(jnp.float32)
        meta_ref[...] = jnp.concatenate(
            [pos, e_flat.astype(jnp.float32), w1, w2, eb5_ref[...]], axis=0)

    @pl.when(j > 0)
    def _expert():
        e = j - 1
        ioE = lax.broadcasted_iota(jnp.int32, (1, E), 1)
        cnt_e = jnp.sum(jnp.where(ioE == e, cnt_s[...], 0))
        er = er_s[...]
        e1, e2 = er[:, 0:1], er[:, 1:2]
        r1, r2 = er[:, 2:3], er[:, 3:4]
        zv = z_s[...]

        for b in range(NBPE):
            @pl.when(cnt_e > b * R)
            def _block(b=b):
                c0 = b * R + lax.broadcasted_iota(jnp.int32, (B, R), 1)
                sel = ((e1 == e) & (r1 == c0)) | ((e2 == e) & (r2 == c0))
                xg = lax.dot_general(sel.astype(jnp.float32), zv,
                                     (((0,), (0,)), ((), ())),
                                     preferred_element_type=jnp.float32)
                h = _gelu(_dott(xg, w1_ref[0]) + b1_ref[0])
                h = _gelu(_dott(h, w2_ref[0]) + b2_ref[0])
                h = _gelu(_dott(h, w3_ref[0]) + b3_ref[0])
                h = _gelu(_dott(h, w4_ref[0]) + b4_ref[0])
                op_ref[b * R:(b + 1) * R, :] = jnp.sum(h * w5_ref[0], axis=1,
                                                       keepdims=True)


def _emap(j):
    return (jnp.maximum(j - 1, 0), 0, 0)


def _mega_call(x, proj_W, proj_b, in_proj_W, in_proj_b, out_proj_W,
               out_proj_b, ln_gamma, ln_beta, gate_W, gate_b, eb5,
               eW1, eb1, eW2, eb2, eW3, eb3, eW4, eb4, eW5):
    def const2(shape):
        return pl.BlockSpec(shape, lambda j: (0, 0))

    return pl.pallas_call(
        _mega_body,
        grid=(1 + E,),
        in_specs=[
            const2((B, D)), const2((D, D)), const2((1, D)),
            const2((3 * D, D)), const2((1, 3 * D)), const2((D, D)),
            const2((1, D)), const2((1, D)), const2((1, D)),
            const2((E, D)), const2((1, E)), const2((E, 1)),
            pl.BlockSpec((1, 1024, 1024), _emap),
            pl.BlockSpec((1, 1, 1024), _emap),
            pl.BlockSpec((1, 512, 1024), _emap),
            pl.BlockSpec((1, 1, 512), _emap),
            pl.BlockSpec((1, 256, 512), _emap),
            pl.BlockSpec((1, 1, 256), _emap),
            pl.BlockSpec((1, 128, 256), _emap),
            pl.BlockSpec((1, 1, 128), _emap),
            pl.BlockSpec((1, 1, 128), _emap),
        ],
        out_specs=(
            pl.BlockSpec((CPE, 1), lambda j: (jnp.maximum(j - 1, 0), 0)),
            pl.BlockSpec((MLEN, 1), lambda j: (0, 0)),
        ),
        out_shape=(
            jax.ShapeDtypeStruct((OPLEN, 1), jnp.float32),
            jax.ShapeDtypeStruct((MLEN, 1), jnp.float32),
        ),
        scratch_shapes=[
            pltpu.VMEM((B, D), jnp.float32),
            pltpu.VMEM((B, 4), jnp.int32),
            pltpu.VMEM((1, E), jnp.int32),
        ],
    )(x, proj_W, proj_b.reshape(1, D), in_proj_W, in_proj_b.reshape(1, 3 * D),
      out_proj_W, out_proj_b.reshape(1, D), ln_gamma.reshape(1, D),
      ln_beta.reshape(1, D), gate_W, gate_b.reshape(1, E), eb5,
      eW1, eb1.reshape(E, 1, 1024), eW2, eb2.reshape(E, 1, 512),
      eW3, eb3.reshape(E, 1, 256), eW4, eb4.reshape(E, 1, 128), eW5)


TPW = B // NW


def _sc_combine_body(op_hbm, meta_hbm, out_hbm, op_v, meta_v, out_v):
    wid = lax.axis_index("s") * SC_NC + lax.axis_index("c")
    base = wid * TPW
    pltpu.sync_copy(op_hbm, op_v)
    pltpu.sync_copy(meta_hbm, meta_v)
    pos1 = meta_v[pl.ds(base, TPW)].astype(jnp.int32)
    pos2 = meta_v[pl.ds(B + base, TPW)].astype(jnp.int32)
    ef1 = meta_v[pl.ds(P + base, TPW)].astype(jnp.int32)
    ef2 = meta_v[pl.ds(P + B + base, TPW)].astype(jnp.int32)
    s1 = plsc.load_gather(op_v, [pos1])
    s2 = plsc.load_gather(op_v, [pos2])
    b1 = plsc.load_gather(meta_v, [(2 * P + 2 * B) + ef1])
    b2 = plsc.load_gather(meta_v, [(2 * P + 2 * B) + ef2])
    w1 = meta_v[pl.ds(2 * P + base, TPW)]
    w2 = meta_v[pl.ds(2 * P + B + base, TPW)]
    x = (s1 + b1) * w1 + (s2 + b2) * w2
    out_v[...] = 1.0 / (1.0 + jnp.exp(-x))
    pltpu.sync_copy(out_v, out_hbm.at[pl.ds(base, TPW)])


def _sc_combine(op, meta):
    mesh = plsc.VectorSubcoreMesh(core_axis_name="c", subcore_axis_name="s")
    return pl.kernel(
        _sc_combine_body,
        out_type=jax.ShapeDtypeStruct((B,), jnp.float32),
        mesh=mesh,
        scratch_types=[
            pltpu.VMEM((OPLEN,), jnp.float32),
            pltpu.VMEM((MLEN,), jnp.float32),
            pltpu.VMEM((TPW,), jnp.float32),
        ],
        compiler_params=pltpu.CompilerParams(needs_layout_passes=False),
    )(op, meta)


def kernel(con_output, proj_W, proj_b, in_proj_W, in_proj_b, out_proj_W,
           out_proj_b, ln_gamma, ln_beta, gate_W, gate_b,
           eW1, eb1, eW2, eb2, eW3, eb3, eW4, eb4, eW5, eb5):
    op, meta = _mega_call(
        con_output, proj_W, proj_b, in_proj_W, in_proj_b, out_proj_W,
        out_proj_b, ln_gamma, ln_beta, gate_W, gate_b, eb5,
        eW1, eb1, eW2, eb2, eW3, eb3, eW4, eb4, eW5)
    return _sc_combine(op.reshape(OPLEN), meta.reshape(MLEN))


# speedup vs baseline: 1.0962x; 1.0962x over previous
import jax
import jax.numpy as jnp
import numpy as np
from jax import lax
from jax.experimental import pallas as pl
from jax.experimental.pallas import tpu as pltpu
from jax.experimental.pallas import tpu_sc as plsc

B = 512
D = 1024
NH = 8
DH = D // NH
E = 16
K = 2
P = B * K
R = 128
CAP = 3072
NBLK = CAP // R
MLEN = 2 * P + 2 * B + E
SC_NC = 2
SC_NS = 16
NW = SC_NC * SC_NS


def _dott(a, b):
    return lax.dot_general(a, b, (((1,), (1,)), ((), ())),
                           preferred_element_type=jnp.float32)


def _dot(a, b):
    return lax.dot_general(a, b, (((1,), (0,)), ((), ())),
                           preferred_element_type=jnp.float32)


def _gelu(x):
    return 0.5 * x * (1.0 + lax.erf(x * np.float32(1.0 / np.sqrt(2.0))))


def _front_body(x_ref, pw_ref, pb_ref, iw_ref, ib_ref, ow_ref, ob_ref,
                lg_ref, lb_ref, gw_ref, gb_ref, eb5_ref,
                z_ref, er_ref, bexp_ref, bact_ref, brow_ref, meta_ref):
    proj = _dott(x_ref[...], pw_ref[...]) + pb_ref[...]
    qkv = _dott(proj, iw_ref[...]) + ib_ref[...]
    heads = []
    for h in range(NH):
        q = qkv[:, h * DH:(h + 1) * DH]
        k = qkv[:, D + h * DH:D + (h + 1) * DH]
        v = qkv[:, 2 * D + h * DH:2 * D + (h + 1) * DH]
        s = _dott(q, k) * np.float32(1.0 / np.sqrt(DH))
        m = jnp.max(s, axis=-1, keepdims=True)
        ex = jnp.exp(s - m)
        attn = ex / jnp.sum(ex, axis=-1, keepdims=True)
        heads.append(_dot(attn, v))
    o = jnp.concatenate(heads, axis=1)
    ao = _dott(o, ow_ref[...]) + ob_ref[...]
    mu = jnp.mean(ao, axis=-1, keepdims=True)
    var = jnp.mean((ao - mu) ** 2, axis=-1, keepdims=True)
    z = (ao - mu) / jnp.sqrt(var + np.float32(1e-5)) * lg_ref[...] + lb_ref[...]
    z_ref[...] = z

    g = _dott(z, gw_ref[...]) + gb_ref[...]
    ioL = lax.broadcasted_iota(jnp.int32, (B, E), 1)
    l1 = jnp.max(g, axis=1, keepdims=True)
    i1 = jnp.min(jnp.where(g == l1, ioL, E), axis=1, keepdims=True)
    gm = jnp.where(ioL == i1, -jnp.inf, g)
    l2 = jnp.max(gm, axis=1, keepdims=True)
    i2 = jnp.min(jnp.where(gm == l2, ioL, E), axis=1, keepdims=True)
    w1 = 1.0 / (1.0 + jnp.exp(l2 - l1))
    w2 = 1.0 - w1

    oh1 = (ioL == i1).astype(jnp.float32)
    oh2 = (ioL == i2).astype(jnp.float32)
    c1 = jnp.sum(oh1, axis=0, keepdims=True)
    counts = (c1 + jnp.sum(oh2, axis=0, keepdims=True)).astype(jnp.int32)
    padded = ((counts + (R - 1)) // R) * R
    mr = lax.broadcasted_iota(jnp.int32, (E, E), 0)
    mc = lax.broadcasted_iota(jnp.int32, (E, E), 1)
    mstrict = (mr < mc).astype(jnp.float32)
    off_row = _dot(padded.astype(jnp.float32), mstrict)
    tr = lax.broadcasted_iota(jnp.int32, (B, B), 0)
    tc = lax.broadcasted_iota(jnp.int32, (B, B), 1)
    tstrict = (tc < tr).astype(jnp.float32)
    rank1 = jnp.sum(_dot(tstrict, oh1) * oh1, axis=1, keepdims=True)
    rank2 = jnp.sum((_dot(tstrict, oh2) + c1) * oh2, axis=1, keepdims=True)
    rank = jnp.concatenate([rank1, rank2], axis=0).astype(jnp.int32)
    er_ref[...] = jnp.concatenate([i1, i2, rank[:B], rank[B:]], axis=1)

    oh = jnp.concatenate([oh1, oh2], axis=0)
    posoff = jnp.sum(oh * off_row, axis=1, keepdims=True)
    pos = posoff + rank.astype(jnp.float32)
    e_flat = jnp.concatenate([i1, i2], axis=0).astype(jnp.float32)
    meta_ref[...] = jnp.concatenate([pos, e_flat, w1, w2, eb5_ref[...]],
                                    axis=0)

    bs = lax.broadcasted_iota(jnp.int32, (NBLK, 1), 0) * R
    offi = off_row.astype(jnp.int32)
    bexp = jnp.sum((offi <= bs).astype(jnp.int32), axis=1, keepdims=True) - 1
    bexp_ref[...] = bexp
    behot = (lax.broadcasted_iota(jnp.int32, (NBLK, E), 1) == bexp)
    offsel = jnp.sum(behot.astype(jnp.float32) * off_row, axis=1,
                     keepdims=True).astype(jnp.int32)
    brow_ref[...] = bs - offsel
    bact_ref[...] = (bs < jnp.sum(padded)).astype(jnp.int32)


def _front_call(x, proj_W, proj_b, in_proj_W, in_proj_b, out_proj_W,
                out_proj_b, ln_gamma, ln_beta, gate_W, gate_b, eb5):
    return pl.pallas_call(
        _front_body,
        out_shape=(
            jax.ShapeDtypeStruct((B, D), jnp.float32),
            jax.ShapeDtypeStruct((B, 4), jnp.int32),
            jax.ShapeDtypeStruct((NBLK, 1), jnp.int32),
            jax.ShapeDtypeStruct((NBLK, 1), jnp.int32),
            jax.ShapeDtypeStruct((NBLK, 1), jnp.int32),
            jax.ShapeDtypeStruct((MLEN, 1), jnp.float32),
        ),
    )(x, proj_W, proj_b.reshape(1, D), in_proj_W, in_proj_b.reshape(1, 3 * D),
      out_proj_W, out_proj_b.reshape(1, D), ln_gamma.reshape(1, D),
      ln_beta.reshape(1, D), gate_W, gate_b.reshape(1, E), eb5)


def _expert_body(be_ref, act_ref, brow_ref, er_ref, z_ref,
                 w1_ref, b1_ref, w2_ref, b2_ref, w3_ref, b3_ref, w4_ref,
                 b4_ref, w5_ref, out_ref):
    j = pl.program_id(0)

    @pl.when(act_ref[j] != 0)
    def _active():
        be = be_ref[j]
        er = er_ref[...]
        e1, e2 = er[:, 0:1], er[:, 1:2]
        r1, r2 = er[:, 2:3], er[:, 3:4]
        c0 = brow_ref[j] + lax.broadcasted_iota(jnp.int32, (B, R), 1)
        sel = ((e1 == be) & (r1 == c0)) | ((e2 == be) & (r2 == c0))
        x = lax.dot_general(sel.astype(jnp.float32), z_ref[...],
                            (((0,), (0,)), ((), ())),
                            preferred_element_type=jnp.float32)
        h = _gelu(_dott(x, w1_ref[0]) + b1_ref[0])
        h = _gelu(_dott(h, w2_ref[0]) + b2_ref[0])
        h = _gelu(_dott(h, w3_ref[0]) + b3_ref[0])
        h = _gelu(_dott(h, w4_ref[0]) + b4_ref[0])
        out_ref[...] = jnp.sum(h * w5_ref[0], axis=1, keepdims=True)


def _expert_call(bexp, bact, brow, er, z, eW1, eb1, eW2, eb2, eW3, eb3, eW4,
                 eb4, eW5):
    def _wmap(j, be, act, brow):
        return (be[j], 0, 0)

    def _wspec(shape):
        return pl.BlockSpec(shape, _wmap)

    grid_spec = pltpu.PrefetchScalarGridSpec(
        num_scalar_prefetch=3,
        grid=(NBLK,),
        in_specs=[
            pl.BlockSpec((B, 4), lambda j, be, act, brow: (0, 0)),
            pl.BlockSpec((B, D), lambda j, be, act, brow: (0, 0)),
            _wspec((1, 1024, 1024)),
            _wspec((1, 1, 1024)),
            _wspec((1, 512, 1024)),
            _wspec((1, 1, 512)),
            _wspec((1, 256, 512)),
            _wspec((1, 1, 256)),
            _wspec((1, 128, 256)),
            _wspec((1, 1, 128)),
            _wspec((1, 1, 128)),
        ],
        out_specs=pl.BlockSpec((R, 1), lambda j, be, act, brow: (j, 0)),
    )
    return pl.pallas_call(
        _expert_body,
        grid_spec=grid_spec,
        out_shape=jax.ShapeDtypeStruct((CAP, 1), jnp.float32),
    )(bexp, bact, brow, er, z,
      eW1, eb1.reshape(E, 1, 1024), eW2, eb2.reshape(E, 1, 512),
      eW3, eb3.reshape(E, 1, 256), eW4, eb4.reshape(E, 1, 128), eW5)


TPW = B // NW


def _sc_combine_body(op_hbm, meta_hbm, out_hbm, op_v, meta_v, out_v):
    wid = lax.axis_index("s") * SC_NC + lax.axis_index("c")
    base = wid * TPW
    pltpu.sync_copy(op_hbm, op_v)
    pltpu.sync_copy(meta_hbm, meta_v)
    pos1 = meta_v[pl.ds(base, TPW)].astype(jnp.int32)
    pos2 = meta_v[pl.ds(B + base, TPW)].astype(jnp.int32)
    ef1 = meta_v[pl.ds(P + base, TPW)].astype(jnp.int32)
    ef2 = meta_v[pl.ds(P + B + base, TPW)].astype(jnp.int32)
    s1 = plsc.load_gather(op_v, [pos1])
    s2 = plsc.load_gather(op_v, [pos2])
    b1 = plsc.load_gather(meta_v, [(2 * P + 2 * B) + ef1])
    b2 = plsc.load_gather(meta_v, [(2 * P + 2 * B) + ef2])
    w1 = meta_v[pl.ds(2 * P + base, TPW)]
    w2 = meta_v[pl.ds(2 * P + B + base, TPW)]
    x = (s1 + b1) * w1 + (s2 + b2) * w2
    out_v[...] = 1.0 / (1.0 + jnp.exp(-x))
    pltpu.sync_copy(out_v, out_hbm.at[pl.ds(base, TPW)])


def _sc_combine(op, meta):
    mesh = plsc.VectorSubcoreMesh(core_axis_name="c", subcore_axis_name="s")
    return pl.kernel(
        _sc_combine_body,
        out_type=jax.ShapeDtypeStruct((B,), jnp.float32),
        mesh=mesh,
        scratch_types=[
            pltpu.VMEM((CAP,), jnp.float32),
            pltpu.VMEM((MLEN,), jnp.float32),
            pltpu.VMEM((TPW,), jnp.float32),
        ],
        compiler_params=pltpu.CompilerParams(needs_layout_passes=False),
    )(op, meta)


def kernel(con_output, proj_W, proj_b, in_proj_W, in_proj_b, out_proj_W,
           out_proj_b, ln_gamma, ln_beta, gate_W, gate_b,
           eW1, eb1, eW2, eb2, eW3, eb3, eW4, eb4, eW5, eb5):
    z, er, bexp, bact, brow, meta = _front_call(
        con_output, proj_W, proj_b, in_proj_W, in_proj_b, out_proj_W,
        out_proj_b, ln_gamma, ln_beta, gate_W, gate_b, eb5)
    op = _expert_call(bexp.reshape(NBLK), bact.reshape(NBLK),
                      brow.reshape(NBLK), er, z,
                      eW1, eb1, eW2, eb2, eW3, eb3, eW4, eb4, eW5)
    return _sc_combine(op.reshape(CAP), meta.reshape(MLEN))


# speedup vs baseline: 1.1037x; 1.0069x over previous
import jax
import jax.numpy as jnp
import numpy as np
from jax import lax
from jax.experimental import pallas as pl
from jax.experimental.pallas import tpu as pltpu
from jax.experimental.pallas import tpu_sc as plsc

B = 512
D = 1024
NH = 8
DH = D // NH
E = 16
K = 2
P = B * K
R = 128
CAP = 3072
NBLK = CAP // R
MLEN = 2 * P + 2 * B + E
SC_NC = 2
SC_NS = 16
NW = SC_NC * SC_NS


def _dott(a, b):
    return lax.dot_general(a, b, (((1,), (1,)), ((), ())),
                           preferred_element_type=jnp.float32)


def _dot(a, b):
    return lax.dot_general(a, b, (((1,), (0,)), ((), ())),
                           preferred_element_type=jnp.float32)


def _gelu(x):
    return 0.5 * x * (1.0 + lax.erf(x * np.float32(1.0 / np.sqrt(2.0))))


def _front_body(x_ref, pw_ref, pb_ref, iw_ref, ib_ref, ow_ref, ob_ref,
                lg_ref, lb_ref, gw_ref, gb_ref, eb5_ref,
                z_ref, er_ref, bexp_ref, bact_ref, brow_ref, meta_ref):
    proj = _dott(x_ref[...], pw_ref[...]) + pb_ref[...]
    qkv = _dott(proj, iw_ref[...]) + ib_ref[...]
    heads = []
    for h in range(NH):
        q = qkv[:, h * DH:(h + 1) * DH]
        k = qkv[:, D + h * DH:D + (h + 1) * DH]
        v = qkv[:, 2 * D + h * DH:2 * D + (h + 1) * DH]
        s = _dott(q, k) * np.float32(1.0 / np.sqrt(DH))
        m = jnp.max(s, axis=-1, keepdims=True)
        ex = jnp.exp(s - m)
        attn = ex / jnp.sum(ex, axis=-1, keepdims=True)
        heads.append(_dot(attn, v))
    o = jnp.concatenate(heads, axis=1)
    ao = _dott(o, ow_ref[...]) + ob_ref[...]
    mu = jnp.mean(ao, axis=-1, keepdims=True)
    var = jnp.mean((ao - mu) ** 2, axis=-1, keepdims=True)
    z = (ao - mu) / jnp.sqrt(var + np.float32(1e-5)) * lg_ref[...] + lb_ref[...]
    z_ref[...] = z

    g = _dott(z, gw_ref[...]) + gb_ref[...]
    ioL = lax.broadcasted_iota(jnp.int32, (B, E), 1)
    l1 = jnp.max(g, axis=1, keepdims=True)
    i1 = jnp.min(jnp.where(g == l1, ioL, E), axis=1, keepdims=True)
    gm = jnp.where(ioL == i1, -jnp.inf, g)
    l2 = jnp.max(gm, axis=1, keepdims=True)
    i2 = jnp.min(jnp.where(gm == l2, ioL, E), axis=1, keepdims=True)
    w1 = 1.0 / (1.0 + jnp.exp(l2 - l1))
    w2 = 1.0 - w1

    oh = jnp.concatenate([ioL == i1, ioL == i2], axis=0).astype(jnp.float32)
    counts = jnp.sum(oh, axis=0, keepdims=True).astype(jnp.int32)
    padded = ((counts + (R - 1)) // R) * R
    mr = lax.broadcasted_iota(jnp.int32, (E, E), 0)
    mc = lax.broadcasted_iota(jnp.int32, (E, E), 1)
    mstrict = (mr < mc).astype(jnp.float32)
    off_row = _dot(padded.astype(jnp.float32), mstrict)
    tr = lax.broadcasted_iota(jnp.int32, (P, P), 0)
    tc = lax.broadcasted_iota(jnp.int32, (P, P), 1)
    tstrict = (tc < tr).astype(jnp.float32)
    rankmat = _dot(tstrict, oh)
    rank = jnp.sum(rankmat * oh, axis=1,
                   keepdims=True).astype(jnp.int32)
    er_ref[...] = jnp.concatenate([i1, i2, rank[:B], rank[B:]], axis=1)

    posoff = jnp.sum(oh * off_row, axis=1, keepdims=True)
    pos = posoff + rank.astype(jnp.float32)
    e_flat = jnp.concatenate([i1, i2], axis=0).astype(jnp.float32)
    meta_ref[...] = jnp.concatenate([pos, e_flat, w1, w2, eb5_ref[...]],
                                    axis=0)

    bs = lax.broadcasted_iota(jnp.int32, (NBLK, 1), 0) * R
    offi = off_row.astype(jnp.int32)
    bexp = jnp.sum((offi <= bs).astype(jnp.int32), axis=1, keepdims=True) - 1
    bexp_ref[...] = bexp
    behot = (lax.broadcasted_iota(jnp.int32, (NBLK, E), 1) == bexp)
    offsel = jnp.sum(behot.astype(jnp.float32) * off_row, axis=1,
                     keepdims=True).astype(jnp.int32)
    brow_ref[...] = bs - offsel
    bact_ref[...] = (bs < jnp.sum(padded)).astype(jnp.int32)


def _front_call(x, proj_W, proj_b, in_proj_W, in_proj_b, out_proj_W,
                out_proj_b, ln_gamma, ln_beta, gate_W, gate_b, eb5):
    return pl.pallas_call(
        _front_body,
        out_shape=(
            jax.ShapeDtypeStruct((B, D), jnp.float32),
            jax.ShapeDtypeStruct((B, 4), jnp.int32),
            jax.ShapeDtypeStruct((NBLK, 1), jnp.int32),
            jax.ShapeDtypeStruct((NBLK, 1), jnp.int32),
            jax.ShapeDtypeStruct((NBLK, 1), jnp.int32),
            jax.ShapeDtypeStruct((MLEN, 1), jnp.float32),
        ),
    )(x, proj_W, proj_b.reshape(1, D), in_proj_W, in_proj_b.reshape(1, 3 * D),
      out_proj_W, out_proj_b.reshape(1, D), ln_gamma.reshape(1, D),
      ln_beta.reshape(1, D), gate_W, gate_b.reshape(1, E), eb5)


def _expert_body(be_ref, act_ref, brow_ref, er_ref, z_ref,
                 w1_ref, b1_ref, w2_ref, b2_ref, w3_ref, b3_ref, w4_ref,
                 b4_ref, w5_ref, out_ref):
    j = pl.program_id(0)

    @pl.when(act_ref[j] != 0)
    def _active():
        be = be_ref[j]
        er = er_ref[...]
        e1, e2 = er[:, 0:1], er[:, 1:2]
        r1, r2 = er[:, 2:3], er[:, 3:4]
        c0 = brow_ref[j] + lax.broadcasted_iota(jnp.int32, (B, R), 1)
        sel = ((e1 == be) & (r1 == c0)) | ((e2 == be) & (r2 == c0))
        x = lax.dot_general(sel.astype(jnp.float32), z_ref[...],
                            (((0,), (0,)), ((), ())),
                            preferred_element_type=jnp.float32)
        h = _gelu(_dott(x, w1_ref[0]) + b1_ref[0])
        h = _gelu(_dott(h, w2_ref[0]) + b2_ref[0])
        h = _gelu(_dott(h, w3_ref[0]) + b3_ref[0])
        h = _gelu(_dott(h, w4_ref[0]) + b4_ref[0])
        out_ref[...] = jnp.sum(h * w5_ref[0], axis=1, keepdims=True)


def _expert_call(bexp, bact, brow, er, z, eW1, eb1, eW2, eb2, eW3, eb3, eW4,
                 eb4, eW5):
    def _wmap(j, be, act, brow):
        return (be[j], 0, 0)

    def _wspec(shape):
        return pl.BlockSpec(shape, _wmap)

    grid_spec = pltpu.PrefetchScalarGridSpec(
        num_scalar_prefetch=3,
        grid=(NBLK,),
        in_specs=[
            pl.BlockSpec((B, 4), lambda j, be, act, brow: (0, 0)),
            pl.BlockSpec((B, D), lambda j, be, act, brow: (0, 0)),
            _wspec((1, 1024, 1024)),
            _wspec((1, 1, 1024)),
            _wspec((1, 512, 1024)),
            _wspec((1, 1, 512)),
            _wspec((1, 256, 512)),
            _wspec((1, 1, 256)),
            _wspec((1, 128, 256)),
            _wspec((1, 1, 128)),
            _wspec((1, 1, 128)),
        ],
        out_specs=pl.BlockSpec((R, 1), lambda j, be, act, brow: (j, 0)),
    )
    return pl.pallas_call(
        _expert_body,
        grid_spec=grid_spec,
        out_shape=jax.ShapeDtypeStruct((CAP, 1), jnp.float32),
    )(bexp, bact, brow, er, z,
      eW1, eb1.reshape(E, 1, 1024), eW2, eb2.reshape(E, 1, 512),
      eW3, eb3.reshape(E, 1, 256), eW4, eb4.reshape(E, 1, 128), eW5)


TPW = B // NW


def _sc_combine_body(op_hbm, meta_hbm, out_hbm, op_v, meta_v, out_v):
    wid = lax.axis_index("s") * SC_NC + lax.axis_index("c")
    base = wid * TPW
    pltpu.sync_copy(op_hbm, op_v)
    pltpu.sync_copy(meta_hbm, meta_v)
    pos1 = meta_v[pl.ds(base, TPW)].astype(jnp.int32)
    pos2 = meta_v[pl.ds(B + base, TPW)].astype(jnp.int32)
    ef1 = meta_v[pl.ds(P + base, TPW)].astype(jnp.int32)
    ef2 = meta_v[pl.ds(P + B + base, TPW)].astype(jnp.int32)
    s1 = plsc.load_gather(op_v, [pos1])
    s2 = plsc.load_gather(op_v, [pos2])
    b1 = plsc.load_gather(meta_v, [(2 * P + 2 * B) + ef1])
    b2 = plsc.load_gather(meta_v, [(2 * P + 2 * B) + ef2])
    w1 = meta_v[pl.ds(2 * P + base, TPW)]
    w2 = meta_v[pl.ds(2 * P + B + base, TPW)]
    x = (s1 + b1) * w1 + (s2 + b2) * w2
    out_v[...] = 1.0 / (1.0 + jnp.exp(-x))
    pltpu.sync_copy(out_v, out_hbm.at[pl.ds(base, TPW)])


def _sc_combine(op, meta):
    mesh = plsc.VectorSubcoreMesh(core_axis_name="c", subcore_axis_name="s")
    return pl.kernel(
        _sc_combine_body,
        out_type=jax.ShapeDtypeStruct((B,), jnp.float32),
        mesh=mesh,
        scratch_types=[
            pltpu.VMEM((CAP,), jnp.float32),
            pltpu.VMEM((MLEN,), jnp.float32),
            pltpu.VMEM((TPW,), jnp.float32),
        ],
        compiler_params=pltpu.CompilerParams(needs_layout_passes=False),
    )(op, meta)


def kernel(con_output, proj_W, proj_b, in_proj_W, in_proj_b, out_proj_W,
           out_proj_b, ln_gamma, ln_beta, gate_W, gate_b,
           eW1, eb1, eW2, eb2, eW3, eb3, eW4, eb4, eW5, eb5):
    z, er, bexp, bact, brow, meta = _front_call(
        con_output, proj_W, proj_b, in_proj_W, in_proj_b, out_proj_W,
        out_proj_b, ln_gamma, ln_beta, gate_W, gate_b, eb5)
    op = _expert_call(bexp.reshape(NBLK), bact.reshape(NBLK),
                      brow.reshape(NBLK), er, z,
                      eW1, eb1, eW2, eb2, eW3, eb3, eW4, eb4, eW5)
    return _sc_combine(op.reshape(CAP), meta.reshape(MLEN))
